# baseline (speedup 1.0000x reference)
import functools

import jax
import jax.numpy as jnp
from jax import lax
from jax.experimental import pallas as pl
from jax.experimental.pallas import tpu as pltpu
from jax.experimental.pallas import tpu_sc as plsc

N = 10000
E = 160000
D_IN = 256
DH = 128
DW = 128
NT = 16
NCH = 80
CK = 128
EPT = NCH * CK
EPAD = NT * EPT
NB = 10
BR = N // NB
RPT = 624
DB = 80


def _prep_body(x1, x2, w1, w2, as1, ad1, as2, ad2,
               h1, h2, asrc1, adst1, asrc2, adst2, mx1, mx2,
               m1s, m2s):
    j = pl.program_id(0)

    def branch(x, w, av, dv, h_o, as_o, ad_o, ms, mx_o):
        hh = jnp.dot(x[...], w[...], preferred_element_type=jnp.float32)
        h_o[...] = hh
        a = lax.dot_general(hh, av[...], (((1,), (1,)), ((), ())),
                            preferred_element_type=jnp.float32)
        as_o[...] = a
        d = lax.dot_general(hh, dv[...], (((1,), (1,)), ((), ())),
                            preferred_element_type=jnp.float32)
        ad_o[...] = d
        m = jnp.max(a)
        ms[0] = jnp.where(j == 0, m, jnp.maximum(ms[0], m))

        @pl.when(j == NB - 1)
        def _():
            mx_o[...] = jnp.full((1, 128), ms[0], jnp.float32)

    branch(x1, w1, as1, ad1, h1, asrc1, adst1, m1s, mx1)
    branch(x2, w2, as2, ad2, h2, asrc2, adst2, m2s, mx2)


def _prep_call(x1, x2, w1, w2, as1, ad1, as2, ad2):
    f32 = jnp.float32
    blk = lambda shp, im: pl.BlockSpec(shp, im)
    row = lambda j: (j, 0)
    fix = lambda j: (0, 0)
    return pl.pallas_call(
        _prep_body,
        grid=(NB,),
        in_specs=[
            blk((BR, D_IN), row), blk((BR, D_IN), row),
            blk((D_IN, DH), fix), blk((D_IN, DH), fix),
            blk((1, DH), fix), blk((1, DH), fix),
            blk((1, DH), fix), blk((1, DH), fix),
        ],
        out_specs=[
            blk((BR, DH), row), blk((BR, DH), row),
            blk((BR, 1), row), blk((BR, 1), row),
            blk((BR, 1), row), blk((BR, 1), row),
            blk((1, 128), fix), blk((1, 128), fix),
        ],
        out_shape=[
            jax.ShapeDtypeStruct((N, DH), f32), jax.ShapeDtypeStruct((N, DH), f32),
            jax.ShapeDtypeStruct((N, 1), f32), jax.ShapeDtypeStruct((N, 1), f32),
            jax.ShapeDtypeStruct((N, 1), f32), jax.ShapeDtypeStruct((N, 1), f32),
            jax.ShapeDtypeStruct((1, 128), f32), jax.ShapeDtypeStruct((1, 128), f32),
        ],
        scratch_shapes=[pltpu.SMEM((1,), f32), pltpu.SMEM((1,), f32)],
    )(x1, x2, w1, w2, as1, ad1, as2, ad2)


def _sc_branch(s, h_hbm, as_hbm, ad_hbm, src_hbm, dst_hbm, num_o,
               den_o, num_sp, den_sp, src_v, dst_v, ex_all, gsem, ssem):
    z16 = jnp.zeros((16,), jnp.float32)
    iota16 = lax.iota(jnp.int32, 16)

    def phase1(asrc_v, adst_v, den_v, ident_v):
        pltpu.sync_copy(as_hbm, asrc_v)
        pltpu.sync_copy(ad_hbm, adst_v)

        def zden(i, _):
            for kk in range(DW // 16):
                den_v[i, pl.ds(kk * 16, 16)] = z16
            return 0
        lax.fori_loop(0, DB, zden, 0)
        for k in range(DB // 16):
            ident_v[0, pl.ds(k * 16, 16)] = k * 16 + iota16

        @pl.when(s == NT - 1)
        def _():
            pltpu.sync_copy(den_v, den_sp)
        plsc.subcore_barrier()

        ma = asrc_v[pl.ds(N, 16)]

        def superchunk(k, _):
            pltpu.sync_copy(src_hbm.at[pl.ds(k * 8, 8), s], src_v)
            pltpu.sync_copy(dst_hbm.at[pl.ds(k * 8, 8), s], dst_v)
            for cc in range(8):
                base_e = s * EPT + (k * 8 + cc) * CK
                for i in range(CK // 16):
                    sidx = src_v[cc, pl.ds(i * 16, 16)]
                    didx = dst_v[cc, pl.ds(i * 16, 16)]
                    av = plsc.load_gather(asrc_v, [sidx])
                    dv = plsc.load_gather(adst_v, [didx])
                    t = av + dv
                    lg = jnp.where(t > 0, t, 0.2 * t)
                    u = ma + dv
                    mp = jnp.where(u > 0, u, 0.2 * u)
                    ex = jnp.exp(lg - mp)
                    gid = base_e + i * 16 + iota16
                    ex = jnp.where(gid < E, ex, 0.0)
                    ex_all[pl.ds((k * 8 + cc) * CK + i * 16, 16)] = ex
                    plsc.addupdate_scatter(
                        den_v,
                        [lax.shift_right_logical(didx, 7), didx & 127], ex)
            return 0
        lax.fori_loop(0, NCH // 8, superchunk, 0)

        pltpu.sync_copy(den_v, den_sp.at[ident_v.at[0]], add=True)

    pl.run_scoped(phase1,
                  pltpu.VMEM((N + 16,), jnp.float32),
                  pltpu.VMEM((N,), jnp.float32),
                  pltpu.VMEM((DB, DW), jnp.float32),
                  pltpu.VMEM((1, DB), jnp.int32))

    def phase2(rows_v):
        def zrow(i, _):
            for kk in range(DW // 16):
                rows_v[0, i, pl.ds(kk * 16, 16)] = z16
            return 0
        lax.fori_loop(0, CK, zrow, 0)

        zb = rows_v.at[0]

        @pl.when(s < NT - 1)
        def _():
            for k in range(4):
                pltpu.sync_copy(zb, num_sp.at[pl.ds(s * RPT + k * 128, 128)])
            pltpu.sync_copy(zb.at[pl.ds(0, 112)],
                            num_sp.at[pl.ds(s * RPT + 512, 112)])

        @pl.when(s == NT - 1)
        def _():
            for k in range(5):
                pltpu.sync_copy(
                    zb, num_sp.at[pl.ds((NT - 1) * RPT + k * 128, 128)])

        plsc.subcore_barrier()

        pltpu.sync_copy(src_hbm.at[pl.ds(0, 1), s], src_v.at[pl.ds(0, 1)])
        pltpu.sync_copy(dst_hbm.at[pl.ds(0, 1), s], dst_v.at[pl.ds(0, 1)])
        pltpu.async_copy(h_hbm.at[src_v.at[0]], rows_v.at[0], gsem.at[0])

        def chunk(ci, _):
            b = ci & 1
            nb = 1 - b
            pltpu.make_async_copy(h_hbm.at[src_v.at[b]], rows_v.at[b],
                                  gsem.at[b]).wait()

            @plsc.parallel_loop(0, CK // 16, unroll=2)
            def scale(g):
                ex16 = ex_all[pl.ds(ci * CK + g * 16, 16)]
                for l in range(16):
                    e = ex16[l]
                    ev = jnp.full((16,), e, dtype=jnp.float32)
                    r = g * 16 + l
                    for kk in range(DH // 16):
                        sl = pl.ds(kk * 16, 16)
                        rows_v[b, r, sl] = rows_v[b, r, sl] * ev

            pltpu.async_copy(rows_v.at[b], num_sp.at[dst_v.at[b]],
                             ssem.at[b], add=True)

            @pl.when(ci + 1 < NCH)
            def _():
                @pl.when(ci >= 1)
                def _():
                    pltpu.make_async_copy(rows_v.at[nb],
                                          num_sp.at[dst_v.at[nb]],
                                          ssem.at[nb]).wait()
                pltpu.sync_copy(src_hbm.at[pl.ds(ci + 1, 1), s],
                                src_v.at[pl.ds(nb, 1)])
                pltpu.sync_copy(dst_hbm.at[pl.ds(ci + 1, 1), s],
                                dst_v.at[pl.ds(nb, 1)])
                pltpu.async_copy(h_hbm.at[src_v.at[nb]], rows_v.at[nb],
                                 gsem.at[nb])
            return 0
        lax.fori_loop(0, NCH, chunk, 0)

        pltpu.make_async_copy(rows_v.at[0], num_sp.at[dst_v.at[0]],
                              ssem.at[0]).wait()
        pltpu.make_async_copy(rows_v.at[1], num_sp.at[dst_v.at[1]],
                              ssem.at[1]).wait()

    pl.run_scoped(phase2, pltpu.VMEM((2, CK, DH), jnp.float32))

    plsc.subcore_barrier()

    base = s * RPT

    @pl.when(s < NT - 1)
    def _():
        pltpu.sync_copy(num_sp.at[pl.ds(base, RPT)], num_o.at[pl.ds(base, RPT)])

    @pl.when(s == NT - 1)
    def _():
        pltpu.sync_copy(num_sp.at[pl.ds((NT - 1) * RPT, 640)],
                        num_o.at[pl.ds((NT - 1) * RPT, 640)])
        pltpu.sync_copy(den_sp, den_o)


def _make_sc_edges():
    f32 = jnp.float32
    mesh = plsc.VectorSubcoreMesh(core_axis_name="c", subcore_axis_name="s",
                                  num_cores=2)

    @functools.partial(
        pl.kernel, mesh=mesh,
        compiler_params=pltpu.CompilerParams(needs_layout_passes=False),
        out_type=[jax.ShapeDtypeStruct((N, DW), f32),
                  jax.ShapeDtypeStruct((DB, DW), f32),
                  jax.ShapeDtypeStruct((N, DW), f32),
                  jax.ShapeDtypeStruct((DB, DW), f32)],
        scratch_types=[
            pltpu.VMEM_SHARED((N, DW), f32),
            pltpu.VMEM_SHARED((DB, DW), f32),
            pltpu.VMEM((8, CK), jnp.int32),
            pltpu.VMEM((8, CK), jnp.int32),
            pltpu.VMEM((EPT,), f32),
            pltpu.SemaphoreType.DMA((2,)),
            pltpu.SemaphoreType.DMA((2,)),
        ],
    )
    def sc_edges(h1, as1, ad1, src1, dst1,
                 h2, as2, ad2, src2, dst2,
                 num1_o, den1_o, num2_o, den2_o,
                 num_sp, den_sp, src_v, dst_v, ex_all, gsem, ssem):
        c = lax.axis_index("c")
        s = lax.axis_index("s")

        @pl.when(c == 0)
        def _():
            _sc_branch(s, h1, as1, ad1, src1, dst1, num1_o, den1_o,
                       num_sp, den_sp, src_v, dst_v, ex_all, gsem, ssem)

        @pl.when(c == 1)
        def _():
            _sc_branch(s, h2, as2, ad2, src2, dst2, num2_o, den2_o,
                       num_sp, den_sp, src_v, dst_v, ex_all, gsem, ssem)

    return sc_edges


def _fin_body(num1, den1, h1, as1, ad1, mx1, num2, den2, h2, as2, ad2, mx2,
              b1, b2, f1pw, f1pb, f2pw, f2pb, f1w, f1b, f2w, f2b, ow, ob,
              o, acc1, acc2):
    j = pl.program_id(0)
    lr = lambda v, a: jnp.where(v > 0, v, a * v)

    def gat(num, den, h, asv, adv, mx, b):
        t = asv[...] + adv[...]
        exs = jnp.exp(lr(t, 0.2) - lr(mx[0, 0] + adv[...], 0.2))
        return (num[...] + exs * h[...]) / (den[...] + exs + 1e-16) + b[...]

    g1 = gat(num1, den1, h1, as1, ad1, mx1, b1)
    p1 = jnp.sum(lr(g1, 0.01), axis=0, keepdims=True)
    g2 = gat(num2, den2, h2, as2, ad2, mx2, b2)
    z2 = lr(lax.dot_general(g2, f2pw[...], (((1,), (1,)), ((), ())),
                            preferred_element_type=jnp.float32) + f2pb[...],
            0.01)
    p2 = jnp.sum(z2, axis=0, keepdims=True)
    acc1[...] = jnp.where(j == 0, p1, acc1[...] + p1)
    acc2[...] = jnp.where(j == 0, p2, acc2[...] + p2)

    @pl.when(j == NB - 1)
    def _():
        mm1 = acc1[...] * (1.0 / N)
        hh1 = lr(lax.dot_general(mm1, f1pw[...], (((1,), (1,)), ((), ())),
                                 preferred_element_type=jnp.float32)
                 + f1pb[...], 0.01)
        mm2 = lr(acc2[...] * (1.0 / N), 0.01)
        xc = jnp.concatenate([hh1, mm2], axis=1)
        y = lr(lax.dot_general(xc, f1w[...], (((1,), (1,)), ((), ())),
                               preferred_element_type=jnp.float32)
               + f1b[...], 0.01)
        y = lr(lax.dot_general(y, f2w[...], (((1,), (1,)), ((), ())),
                               preferred_element_type=jnp.float32)
               + f2b[...], 0.01)
        v = jnp.sum(y * ow[...], axis=1, keepdims=True) + ob[...]
        o[...] = 1.0 / (1.0 + jnp.exp(-v))


def _fin_call(num1, den1, h1, as1, ad1, mx1, num2, den2, h2, as2, ad2, mx2,
              b1, b2, f1pw, f1pb, f2pw, f2pb, f1w, f1b, f2w, f2b, ow, ob):
    f32 = jnp.float32
    blk = lambda shp, im: pl.BlockSpec(shp, im)
    row = lambda j: (j, 0)
    fix = lambda j: (0, 0)
    node = [blk((BR, DH), row), blk((BR, 1), row), blk((BR, DH), row),
            blk((BR, 1), row), blk((BR, 1), row), blk((1, 128), fix)]
    return pl.pallas_call(
        _fin_body,
        grid=(NB,),
        in_specs=node + node + [
            blk((1, DH), fix), blk((1, DH), fix),
            blk((DH, DH), fix), blk((1, DH), fix),
            blk((DH, DH), fix), blk((1, DH), fix),
            blk((256, 256), fix), blk((1, 256), fix),
            blk((64, 256), fix), blk((1, 64), fix),
            blk((1, 64), fix), blk((1, 1), fix),
        ],
        out_specs=blk((1, 1), fix),
        out_shape=jax.ShapeDtypeStruct((1, 1), f32),
        scratch_shapes=[pltpu.VMEM((1, DH), f32), pltpu.VMEM((1, DH), f32)],
    )(num1, den1, h1, as1, ad1, mx1, num2, den2, h2, as2, ad2, mx2,
      b1, b2, f1pw, f1pb, f2pw, f2pb, f1w, f1b, f2w, f2b, ow, ob)


def kernel(x1, edge_index1, x2, edge_index2,
           W1, a_src1, a_dst1, b1, W2, a_src2, a_dst2, b2,
           fc1p_w, fc1p_b, fc2p_w, fc2p_b,
           fc1_w, fc1_b, fc2_w, fc2_b, out_w, out_b):
    h1, h2, asrc1, adst1, asrc2, adst2, mx1, mx2 = _prep_call(
        x1, x2, W1, W2,
        a_src1.reshape(1, DH), a_dst1.reshape(1, DH),
        a_src2.reshape(1, DH), a_dst2.reshape(1, DH))

    pad = jnp.zeros((EPAD - E,), jnp.int32)

    def elay(e):
        return jnp.transpose(
            jnp.concatenate([e, pad]).reshape(NT, NCH, CK), (1, 0, 2))

    src1 = elay(edge_index1[0])
    dst1 = elay(edge_index1[1])
    src2 = elay(edge_index2[0])
    dst2 = elay(edge_index2[1])

    as1c = jnp.concatenate([asrc1.reshape(N), mx1[0, :16]])
    as2c = jnp.concatenate([asrc2.reshape(N), mx2[0, :16]])
    num1, den1, num2, den2 = _make_sc_edges()(
        h1, as1c, adst1.reshape(N), src1, dst1,
        h2, as2c, adst2.reshape(N), src2, dst2)

    den1 = den1.reshape(DB * DW)[:N].reshape(N, 1)
    den2 = den2.reshape(DB * DW)[:N].reshape(N, 1)
    o = _fin_call(num1, den1, h1, asrc1, adst1, mx1,
                  num2, den2, h2, asrc2, adst2, mx2,
                  b1.reshape(1, DH), b2.reshape(1, DH),
                  fc1p_w, fc1p_b.reshape(1, DH),
                  fc2p_w, fc2p_b.reshape(1, DH),
                  fc1_w, fc1_b.reshape(1, 256),
                  fc2_w, fc2_b.reshape(1, 64),
                  out_w, out_b.reshape(1, 1))
    return o


# speedup vs baseline: 1.3719x; 1.3719x over previous
import functools

import jax
import jax.numpy as jnp
from jax import lax
from jax.experimental import pallas as pl
from jax.experimental.pallas import tpu as pltpu
from jax.experimental.pallas import tpu_sc as plsc

N = 10000
E = 160000
D_IN = 256
DH = 128
DW = 128
NT = 16
NCH = 80
CK = 128
EPT = NCH * CK
EPAD = NT * EPT
NB = 10
BR = N // NB
RPT = 624
DB = 80


def _prep_body(x1, x2, w1, w2, as1, ad1, as2, ad2,
               h1, h2, asrc1, adst1, asrc2, adst2, mx1, mx2,
               m1s, m2s):
    j = pl.program_id(0)

    def branch(x, w, av, dv, h_o, as_o, ad_o, ms, mx_o):
        hh = jnp.dot(x[...], w[...], preferred_element_type=jnp.float32)
        h_o[...] = hh
        a = lax.dot_general(hh, av[...], (((1,), (1,)), ((), ())),
                            preferred_element_type=jnp.float32)
        as_o[...] = a
        d = lax.dot_general(hh, dv[...], (((1,), (1,)), ((), ())),
                            preferred_element_type=jnp.float32)
        ad_o[...] = d
        m = jnp.max(a)
        ms[0] = jnp.where(j == 0, m, jnp.maximum(ms[0], m))

        @pl.when(j == NB - 1)
        def _():
            mx_o[...] = jnp.full((1, 128), ms[0], jnp.float32)

    branch(x1, w1, as1, ad1, h1, asrc1, adst1, m1s, mx1)
    branch(x2, w2, as2, ad2, h2, asrc2, adst2, m2s, mx2)


def _prep_call(x1, x2, w1, w2, as1, ad1, as2, ad2):
    f32 = jnp.float32
    blk = lambda shp, im: pl.BlockSpec(shp, im)
    row = lambda j: (j, 0)
    fix = lambda j: (0, 0)
    return pl.pallas_call(
        _prep_body,
        grid=(NB,),
        in_specs=[
            blk((BR, D_IN), row), blk((BR, D_IN), row),
            blk((D_IN, DH), fix), blk((D_IN, DH), fix),
            blk((1, DH), fix), blk((1, DH), fix),
            blk((1, DH), fix), blk((1, DH), fix),
        ],
        out_specs=[
            blk((BR, DH), row), blk((BR, DH), row),
            blk((BR, 1), row), blk((BR, 1), row),
            blk((BR, 1), row), blk((BR, 1), row),
            blk((1, 128), fix), blk((1, 128), fix),
        ],
        out_shape=[
            jax.ShapeDtypeStruct((N, DH), f32), jax.ShapeDtypeStruct((N, DH), f32),
            jax.ShapeDtypeStruct((N, 1), f32), jax.ShapeDtypeStruct((N, 1), f32),
            jax.ShapeDtypeStruct((N, 1), f32), jax.ShapeDtypeStruct((N, 1), f32),
            jax.ShapeDtypeStruct((1, 128), f32), jax.ShapeDtypeStruct((1, 128), f32),
        ],
        scratch_shapes=[pltpu.SMEM((1,), f32), pltpu.SMEM((1,), f32)],
    )(x1, x2, w1, w2, as1, ad1, as2, ad2)


def _sc_branch(s, h_hbm, as_hbm, ad_hbm, src_hbm, dst_hbm, num_o,
               den_o, num_sp, den_sp, asrc_v, adst_v, src_v, dst_v, rows_v,
               ex_v, den_v, ident_v, gsem):
    z16 = jnp.zeros((16,), jnp.float32)
    iota16 = lax.iota(jnp.int32, 16)

    pltpu.sync_copy(as_hbm, asrc_v)
    pltpu.sync_copy(ad_hbm, adst_v)

    def zrow(i, _):
        for kk in range(DW // 16):
            rows_v[i, pl.ds(kk * 16, 16)] = z16
        return 0
    lax.fori_loop(0, CK, zrow, 0)

    def zden(i, _):
        for kk in range(DW // 16):
            den_v[i, pl.ds(kk * 16, 16)] = z16
        return 0
    lax.fori_loop(0, DB, zden, 0)
    for k in range(DB // 16):
        ident_v[0, pl.ds(k * 16, 16)] = k * 16 + iota16

    @pl.when(s < NT - 1)
    def _():
        for k in range(4):
            pltpu.sync_copy(rows_v, num_sp.at[pl.ds(s * RPT + k * 128, 128)])
        pltpu.sync_copy(rows_v.at[pl.ds(0, 112)],
                        num_sp.at[pl.ds(s * RPT + 512, 112)])

    @pl.when(s == NT - 1)
    def _():
        for k in range(5):
            pltpu.sync_copy(rows_v,
                            num_sp.at[pl.ds((NT - 1) * RPT + k * 128, 128)])
        pltpu.sync_copy(rows_v.at[pl.ds(0, DB)], den_sp)

    plsc.subcore_barrier()

    ma = asrc_v[pl.ds(N, 16)]

    def compute_ex(base_e, cc):
        for i in range(CK // 16):
            sidx = src_v[cc, pl.ds(i * 16, 16)]
            didx = dst_v[cc, pl.ds(i * 16, 16)]
            av = plsc.load_gather(asrc_v, [sidx])
            dv = plsc.load_gather(adst_v, [didx])
            t = av + dv
            lg = jnp.where(t > 0, t, 0.2 * t)
            u = ma + dv
            mp = jnp.where(u > 0, u, 0.2 * u)
            ex = jnp.exp(lg - mp)
            gid = base_e + i * 16 + iota16
            ex = jnp.where(gid < E, ex, 0.0)
            ex_v[pl.ds(i * 16, 16)] = ex
            plsc.addupdate_scatter(
                den_v, [lax.shift_right_logical(didx, 7), didx & 127], ex)

    def superchunk(k, _):
        pltpu.sync_copy(src_hbm.at[pl.ds((s * NCH + k * 8), 8)], src_v)
        pltpu.sync_copy(dst_hbm.at[pl.ds((s * NCH + k * 8), 8)], dst_v)

        def chunk(cc, _):
            gat = pltpu.async_copy(h_hbm.at[src_v.at[cc]], rows_v,
                                   gsem.at[0])
            base_e = (s * NCH + k * 8 + cc) * CK
            compute_ex(base_e, cc)
            gat.wait()

            @plsc.parallel_loop(0, CK // 16, unroll=2)
            def scale(g):
                ex16 = ex_v[pl.ds(g * 16, 16)]
                for l in range(16):
                    e = ex16[l]
                    ev = jnp.full((16,), e, dtype=jnp.float32)
                    r = g * 16 + l
                    for kk in range(DH // 16):
                        sl = pl.ds(kk * 16, 16)
                        rows_v[r, sl] = rows_v[r, sl] * ev

            pltpu.sync_copy(rows_v, num_sp.at[dst_v.at[cc]], add=True)
            return 0
        lax.fori_loop(0, 8, chunk, 0)
        return 0
    lax.fori_loop(0, NCH // 8, superchunk, 0)

    pltpu.sync_copy(den_v, den_sp.at[ident_v.at[0]], add=True)
    plsc.subcore_barrier()

    base = s * RPT

    @pl.when(s < NT - 1)
    def _():
        pltpu.sync_copy(num_sp.at[pl.ds(base, RPT)], num_o.at[pl.ds(base, RPT)])

    @pl.when(s == NT - 1)
    def _():
        pltpu.sync_copy(num_sp.at[pl.ds((NT - 1) * RPT, 640)],
                        num_o.at[pl.ds((NT - 1) * RPT, 640)])
        pltpu.sync_copy(den_sp, den_o)


def _make_sc_edges():
    f32 = jnp.float32
    mesh = plsc.VectorSubcoreMesh(core_axis_name="c", subcore_axis_name="s",
                                  num_cores=2)

    @functools.partial(
        pl.kernel, mesh=mesh,
        compiler_params=pltpu.CompilerParams(needs_layout_passes=False),
        out_type=[jax.ShapeDtypeStruct((N, DW), f32),
                  jax.ShapeDtypeStruct((DB, DW), f32),
                  jax.ShapeDtypeStruct((N, DW), f32),
                  jax.ShapeDtypeStruct((DB, DW), f32)],
        scratch_types=[
            pltpu.VMEM_SHARED((N, DW), f32),
            pltpu.VMEM_SHARED((DB, DW), f32),
            pltpu.VMEM((N + 16,), f32),
            pltpu.VMEM((N,), f32),
            pltpu.VMEM((8, CK), jnp.int32),
            pltpu.VMEM((8, CK), jnp.int32),
            pltpu.VMEM((CK, DH), f32),
            pltpu.VMEM((CK,), f32),
            pltpu.VMEM((DB, DW), f32),
            pltpu.VMEM((1, DB), jnp.int32),
            pltpu.SemaphoreType.DMA((2,)),
        ],
    )
    def sc_edges(h1, as1, ad1, src1, dst1,
                 h2, as2, ad2, src2, dst2,
                 num1_o, den1_o, num2_o, den2_o,
                 num_sp, den_sp, asrc_v, adst_v, src_v, dst_v, rows_v,
                 ex_v, den_v, ident_v, gsem):
        c = lax.axis_index("c")
        s = lax.axis_index("s")

        @pl.when(c == 0)
        def _():
            _sc_branch(s, h1, as1, ad1, src1, dst1, num1_o, den1_o,
                       num_sp, den_sp, asrc_v, adst_v, src_v, dst_v, rows_v,
                       ex_v, den_v, ident_v, gsem)

        @pl.when(c == 1)
        def _():
            _sc_branch(s, h2, as2, ad2, src2, dst2, num2_o, den2_o,
                       num_sp, den_sp, asrc_v, adst_v, src_v, dst_v, rows_v,
                       ex_v, den_v, ident_v, gsem)

    return sc_edges


def _fin_body(num1, den1, h1, as1, ad1, mx1, num2, den2, h2, as2, ad2, mx2,
              b1, b2, f1pw, f1pb, f2pw, f2pb, f1w, f1b, f2w, f2b, ow, ob,
              o, acc1, acc2):
    j = pl.program_id(0)
    lr = lambda v, a: jnp.where(v > 0, v, a * v)

    def gat(num, den, h, asv, adv, mx, b):
        t = asv[...] + adv[...]
        exs = jnp.exp(lr(t, 0.2) - lr(mx[0, 0] + adv[...], 0.2))
        return (num[...] + exs * h[...]) / (den[...] + exs + 1e-16) + b[...]

    g1 = gat(num1, den1, h1, as1, ad1, mx1, b1)
    p1 = jnp.sum(lr(g1, 0.01), axis=0, keepdims=True)
    g2 = gat(num2, den2, h2, as2, ad2, mx2, b2)
    z2 = lr(lax.dot_general(g2, f2pw[...], (((1,), (1,)), ((), ())),
                            preferred_element_type=jnp.float32) + f2pb[...],
            0.01)
    p2 = jnp.sum(z2, axis=0, keepdims=True)
    acc1[...] = jnp.where(j == 0, p1, acc1[...] + p1)
    acc2[...] = jnp.where(j == 0, p2, acc2[...] + p2)

    @pl.when(j == NB - 1)
    def _():
        mm1 = acc1[...] * (1.0 / N)
        hh1 = lr(lax.dot_general(mm1, f1pw[...], (((1,), (1,)), ((), ())),
                                 preferred_element_type=jnp.float32)
                 + f1pb[...], 0.01)
        mm2 = lr(acc2[...] * (1.0 / N), 0.01)
        xc = jnp.concatenate([hh1, mm2], axis=1)
        y = lr(lax.dot_general(xc, f1w[...], (((1,), (1,)), ((), ())),
                               preferred_element_type=jnp.float32)
               + f1b[...], 0.01)
        y = lr(lax.dot_general(y, f2w[...], (((1,), (1,)), ((), ())),
                               preferred_element_type=jnp.float32)
               + f2b[...], 0.01)
        v = jnp.sum(y * ow[...], axis=1, keepdims=True) + ob[...]
        o[...] = 1.0 / (1.0 + jnp.exp(-v))


def _fin_call(num1, den1, h1, as1, ad1, mx1, num2, den2, h2, as2, ad2, mx2,
              b1, b2, f1pw, f1pb, f2pw, f2pb, f1w, f1b, f2w, f2b, ow, ob):
    f32 = jnp.float32
    blk = lambda shp, im: pl.BlockSpec(shp, im)
    row = lambda j: (j, 0)
    fix = lambda j: (0, 0)
    node = [blk((BR, DH), row), blk((BR, 1), row), blk((BR, DH), row),
            blk((BR, 1), row), blk((BR, 1), row), blk((1, 128), fix)]
    return pl.pallas_call(
        _fin_body,
        grid=(NB,),
        in_specs=node + node + [
            blk((1, DH), fix), blk((1, DH), fix),
            blk((DH, DH), fix), blk((1, DH), fix),
            blk((DH, DH), fix), blk((1, DH), fix),
            blk((256, 256), fix), blk((1, 256), fix),
            blk((64, 256), fix), blk((1, 64), fix),
            blk((1, 64), fix), blk((1, 1), fix),
        ],
        out_specs=blk((1, 1), fix),
        out_shape=jax.ShapeDtypeStruct((1, 1), f32),
        scratch_shapes=[pltpu.VMEM((1, DH), f32), pltpu.VMEM((1, DH), f32)],
    )(num1, den1, h1, as1, ad1, mx1, num2, den2, h2, as2, ad2, mx2,
      b1, b2, f1pw, f1pb, f2pw, f2pb, f1w, f1b, f2w, f2b, ow, ob)


def kernel(x1, edge_index1, x2, edge_index2,
           W1, a_src1, a_dst1, b1, W2, a_src2, a_dst2, b2,
           fc1p_w, fc1p_b, fc2p_w, fc2p_b,
           fc1_w, fc1_b, fc2_w, fc2_b, out_w, out_b):
    h1, h2, asrc1, adst1, asrc2, adst2, mx1, mx2 = _prep_call(
        x1, x2, W1, W2,
        a_src1.reshape(1, DH), a_dst1.reshape(1, DH),
        a_src2.reshape(1, DH), a_dst2.reshape(1, DH))

    pad = jnp.zeros((EPAD - E,), jnp.int32)

    def elay(e):
        return jnp.concatenate([e, pad]).reshape(NT * NCH, CK)

    src1 = elay(edge_index1[0])
    dst1 = elay(edge_index1[1])
    src2 = elay(edge_index2[0])
    dst2 = elay(edge_index2[1])

    as1c = jnp.concatenate([asrc1.reshape(N), mx1[0, :16]])
    as2c = jnp.concatenate([asrc2.reshape(N), mx2[0, :16]])
    num1, den1, num2, den2 = _make_sc_edges()(
        h1, as1c, adst1.reshape(N), src1, dst1,
        h2, as2c, adst2.reshape(N), src2, dst2)

    den1 = den1.reshape(DB * DW)[:N].reshape(N, 1)
    den2 = den2.reshape(DB * DW)[:N].reshape(N, 1)
    o = _fin_call(num1, den1, h1, asrc1, adst1, mx1,
                  num2, den2, h2, asrc2, adst2, mx2,
                  b1.reshape(1, DH), b2.reshape(1, DH),
                  fc1p_w, fc1p_b.reshape(1, DH),
                  fc2p_w, fc2p_b.reshape(1, DH),
                  fc1_w, fc1_b.reshape(1, 256),
                  fc2_w, fc2_b.reshape(1, 64),
                  out_w, out_b.reshape(1, 1))
    return o
